# Initial kernel scaffold; baseline (speedup 1.0000x reference)
import jax
import jax.numpy as jnp
from jax.experimental import pallas as pl


def kernel(x, hyperedge_index, W_lin, b_lin, W_sheaf, b_sheaf, W_conv, b_conv, bias):
    raise NotImplementedError("write your pallas kernel here")


# speedup vs baseline: 14.5180x; 14.5180x over previous
import functools

import jax
import jax.numpy as jnp
from jax import lax
from jax.experimental import pallas as pl
from jax.experimental.pallas import tpu as pltpu
from jax.experimental.pallas import tpu_sc as plsc

N = 10000
M = 10000
E = 320000
D = 4
HID = 32
F = 128
NC, NS, L = 2, 16, 16
NW = NC * NS
CB = 128
NCHUNK = E // CB
RPAD = 10112
RPT = RPAD // NS

_MESH = plsc.VectorSubcoreMesh(
    core_axis_name="c", subcore_axis_name="s", num_cores=NC, num_subcores=NS)


def _prep_body(x_ref, wl_ref, bl_ref, wb_ref, bb_ref, wp_ref, bp_ref,
               wg_ref, bg_ref, xl_ref, pa_ref, ga_ref):
    h = jnp.dot(x_ref[...], wl_ref[...], preferred_element_type=jnp.float32)
    h = h + bl_ref[...]
    xl_ref[...] = jnp.dot(h, wb_ref[...],
                          preferred_element_type=jnp.float32) + bb_ref[...]
    pa_ref[...] = jnp.dot(h, wp_ref[...],
                          preferred_element_type=jnp.float32) + bp_ref[...]
    ga_ref[...] = jnp.dot(h, wg_ref[...],
                          preferred_element_type=jnp.float32) + bg_ref[...]


def _prep(x, W_lin, b_lin, W_big, b_big, Wp, bp, Wg, bg):
    BR = 2000
    grid = (N // BR,)
    row_spec = pl.BlockSpec((BR, F), lambda i: (i, 0))
    w_spec = pl.BlockSpec((F, F), lambda i: (0, 0))
    w16_spec = pl.BlockSpec((F, L), lambda i: (0, 0))
    b_spec = pl.BlockSpec((1, F), lambda i: (0, 0))
    b16_spec = pl.BlockSpec((1, L), lambda i: (0, 0))
    out16_spec = pl.BlockSpec((BR, L), lambda i: (i, 0))
    return pl.pallas_call(
        _prep_body,
        grid=grid,
        in_specs=[row_spec, w_spec, b_spec, w_spec, b_spec, w16_spec,
                  b16_spec, w16_spec, b16_spec],
        out_specs=[row_spec, out16_spec, out16_spec],
        out_shape=[
            jax.ShapeDtypeStruct((N, F), jnp.float32),
            jax.ShapeDtypeStruct((N, L), jnp.float32),
            jax.ShapeDtypeStruct((N, L), jnp.float32),
        ],
    )(x, W_lin, b_lin.reshape(1, F), W_big, b_big.reshape(1, F),
      Wp, bp.reshape(1, L), Wg, bg.reshape(1, L))


def _sc_countsum_body(row_hbm, col_hbm, garr_hbm, qc_out, deg_out,
                      ridx, cidx, gbuf, qc_sh, deg_sh, sem):
    c = lax.axis_index("c")
    s = lax.axis_index("s")
    gwid = c * NS + s
    base = s * RPT

    def z_body(i, _):
        gbuf[i, :] = jnp.zeros((L,), jnp.float32)
        return 0
    lax.fori_loop(0, CB, z_body, 0)
    for t in range(RPT // CB):
        pltpu.sync_copy(gbuf, qc_sh.at[pl.ds(base + t * CB, CB)])
        pltpu.sync_copy(gbuf, deg_sh.at[pl.ds(base + t * CB, CB)])
    rem = RPT % CB
    pltpu.sync_copy(gbuf.at[pl.ds(0, rem)],
                    qc_sh.at[pl.ds(base + (RPT // CB) * CB, rem)])
    pltpu.sync_copy(gbuf.at[pl.ds(0, rem)],
                    deg_sh.at[pl.ds(base + (RPT // CB) * CB, rem)])
    plsc.subcore_barrier()

    nj = (NCHUNK + NW - 1 - gwid) // NW

    def chunk(jj, _):
        off = (gwid + jj * NW) * CB
        pltpu.sync_copy(row_hbm.at[pl.ds(off, CB)], ridx)
        pltpu.sync_copy(col_hbm.at[pl.ds(off, CB)], cidx)
        pltpu.async_copy(garr_hbm.at[ridx], gbuf, sem).wait()
        pltpu.sync_copy(gbuf, qc_sh.at[cidx], add=True)
        pltpu.sync_copy(gbuf, deg_sh.at[ridx], add=True)
        return 0
    lax.fori_loop(0, nj, chunk, 0)
    plsc.subcore_barrier()
    pltpu.sync_copy(qc_sh.at[pl.ds(base, RPT)],
                    qc_out.at[c, pl.ds(base, RPT)])
    pltpu.sync_copy(deg_sh.at[pl.ds(base, RPT)],
                    deg_out.at[c, pl.ds(base, RPT)])


def _sc_countsum(row, col, garr):
    return pl.kernel(
        _sc_countsum_body,
        out_type=[
            jax.ShapeDtypeStruct((NC, RPAD, L), jnp.float32),
            jax.ShapeDtypeStruct((NC, RPAD, L), jnp.float32),
        ],
        mesh=_MESH,
        compiler_params=pltpu.CompilerParams(use_tc_tiling_on_sc=False),
        scratch_types=[
            pltpu.VMEM((CB,), jnp.int32),
            pltpu.VMEM((CB,), jnp.int32),
            pltpu.VMEM((CB, L), jnp.float32),
            pltpu.VMEM_SHARED((RPAD, L), jnp.float32),
            pltpu.VMEM_SHARED((RPAD, L), jnp.float32),
            pltpu.SemaphoreType.DMA,
        ],
    )(row, col, garr)


def _mid1_body(qc_ref, qm_ref):
    qc = qc_ref[0] + qc_ref[1]
    cnt = qc[:, 4:5]
    inv = 1.0 / jnp.maximum(cnt, 1.0)
    binv = jnp.where(cnt > 0, inv, 0.0)
    li = lax.broadcasted_iota(jnp.int32, qc.shape, 1)
    qm_ref[...] = jnp.where(li == 4, binv, qc * inv)


def _mid1(qc_parts):
    BR = 2528
    return pl.pallas_call(
        _mid1_body,
        grid=(RPAD // BR,),
        in_specs=[pl.BlockSpec((NC, BR, L), lambda i: (0, i, 0))],
        out_specs=pl.BlockSpec((BR, L), lambda i: (i, 0)),
        out_shape=jax.ShapeDtypeStruct((RPAD, L), jnp.float32),
    )(qc_parts)


def _sc_alpha_body(row_hbm, col_hbm, parr_hbm, qm_hbm, alpha_out,
                   ridx, cidx, pbuf, qbuf, sem):
    c = lax.axis_index("c")
    s = lax.axis_index("s")
    gwid = c * NS + s
    nj = (NCHUNK + NW - 1 - gwid) // NW

    def chunk(jj, _):
        off = (gwid + jj * NW) * CB
        pltpu.sync_copy(row_hbm.at[pl.ds(off, CB)], ridx)
        pltpu.sync_copy(col_hbm.at[pl.ds(off, CB)], cidx)
        pltpu.async_copy(parr_hbm.at[ridx], pbuf, sem).wait()
        pltpu.async_copy(qm_hbm.at[cidx], qbuf, sem).wait()

        def e_body(i, _):
            t = pbuf[i, :] + qbuf[i, :]
            t = jnp.minimum(jnp.maximum(t, -15.0), 15.0)
            ex = jnp.exp(t + t)
            pbuf[i, :] = (ex - 1.0) / (ex + 1.0)
            return 0
        lax.fori_loop(0, CB, e_body, 0)
        pltpu.sync_copy(pbuf, alpha_out.at[pl.ds(off, CB)])
        return 0
    lax.fori_loop(0, nj, chunk, 0)


def _sc_alpha(row, col, parr, qmean):
    return pl.kernel(
        _sc_alpha_body,
        out_type=jax.ShapeDtypeStruct((E, L), jnp.float32),
        mesh=_MESH,
        compiler_params=pltpu.CompilerParams(use_tc_tiling_on_sc=False),
        scratch_types=[
            pltpu.VMEM((CB,), jnp.int32),
            pltpu.VMEM((CB,), jnp.int32),
            pltpu.VMEM((CB, L), jnp.float32),
            pltpu.VMEM((CB, L), jnp.float32),
            pltpu.SemaphoreType.DMA,
        ],
    )(row, col, parr, qmean)


def _sc_pass_body(gkey_hbm, skey_hbm, alpha_hbm, src_hbm, acc_out,
                  gidx, sidx, xbuf, abuf, acc_sh, sem):
    c = lax.axis_index("c")
    s = lax.axis_index("s")
    gwid = c * NS + s
    base = s * RPT

    def z_body(i, _):
        for jj in range(F // L):
            xbuf[i, pl.ds(jj * L, L)] = jnp.zeros((L,), jnp.float32)
        return 0
    lax.fori_loop(0, CB, z_body, 0)
    for t in range(RPT // CB):
        pltpu.sync_copy(xbuf, acc_sh.at[pl.ds(base + t * CB, CB)])
    rem = RPT % CB
    pltpu.sync_copy(xbuf.at[pl.ds(0, rem)],
                    acc_sh.at[pl.ds(base + (RPT // CB) * CB, rem)])
    plsc.subcore_barrier()

    nj = (NCHUNK + NW - 1 - gwid) // NW

    def chunk(jj, _):
        off = (gwid + jj * NW) * CB
        pltpu.sync_copy(gkey_hbm.at[pl.ds(off, CB)], gidx)
        pltpu.sync_copy(skey_hbm.at[pl.ds(off, CB)], sidx)
        pltpu.async_copy(src_hbm.at[gidx], xbuf, sem).wait()
        pltpu.sync_copy(alpha_hbm.at[pl.ds(off, CB)], abuf)

        def e_body(i, _):
            av = abuf[i, :]
            for k in range(D):
                a = av[k]
                for h2 in range(2):
                    jj2 = 2 * k + h2
                    xbuf[i, pl.ds(jj2 * L, L)] = \
                        xbuf[i, pl.ds(jj2 * L, L)] * a
            return 0
        lax.fori_loop(0, CB, e_body, 0)
        pltpu.sync_copy(xbuf, acc_sh.at[sidx], add=True)
        return 0
    lax.fori_loop(0, nj, chunk, 0)
    plsc.subcore_barrier()
    pltpu.sync_copy(acc_sh.at[pl.ds(base, RPT)],
                    acc_out.at[c, pl.ds(base, RPT)])


def _sc_pass(gather_key, scatter_key, alpha, src):
    return pl.kernel(
        _sc_pass_body,
        out_type=jax.ShapeDtypeStruct((NC, RPAD, F), jnp.float32),
        mesh=_MESH,
        compiler_params=pltpu.CompilerParams(use_tc_tiling_on_sc=False),
        scratch_types=[
            pltpu.VMEM((CB,), jnp.int32),
            pltpu.VMEM((CB,), jnp.int32),
            pltpu.VMEM((CB, F), jnp.float32),
            pltpu.VMEM((CB, L), jnp.float32),
            pltpu.VMEM_SHARED((RPAD, F), jnp.float32),
            pltpu.SemaphoreType.DMA,
        ],
    )(gather_key, scatter_key, alpha, src)


def _mid2_body(m_ref, qm_ref, out_ref):
    binv = qm_ref[...][:, 4:5]
    out_ref[...] = (m_ref[0] + m_ref[1]) * binv


def _mid2(m_parts, qmean):
    BR = 2528
    return pl.pallas_call(
        _mid2_body,
        grid=(RPAD // BR,),
        in_specs=[pl.BlockSpec((NC, BR, F), lambda i: (0, i, 0)),
                  pl.BlockSpec((BR, L), lambda i: (i, 0))],
        out_specs=pl.BlockSpec((BR, F), lambda i: (i, 0)),
        out_shape=jax.ShapeDtypeStruct((RPAD, F), jnp.float32),
    )(m_parts, qmean)


def _final_body(y_ref, deg_ref, bias_ref, out_ref):
    deg = deg_ref[0][:, 4:5] + deg_ref[1][:, 4:5]
    dinv = jnp.where(deg > 0, 1.0 / jnp.maximum(deg, 1.0), 0.0)
    out_ref[...] = (y_ref[0] + y_ref[1]) * dinv + bias_ref[...]


def _final(y_parts, deg_parts, bias128):
    BR = 2528
    return pl.pallas_call(
        _final_body,
        grid=(RPAD // BR,),
        in_specs=[pl.BlockSpec((NC, BR, F), lambda i: (0, i, 0)),
                  pl.BlockSpec((NC, BR, L), lambda i: (0, i, 0)),
                  pl.BlockSpec((1, F), lambda i: (0, 0))],
        out_specs=pl.BlockSpec((BR, F), lambda i: (i, 0)),
        out_shape=jax.ShapeDtypeStruct((RPAD, F), jnp.float32),
    )(y_parts, deg_parts, bias128)


def kernel(x, hyperedge_index, W_lin, b_lin, W_sheaf, b_sheaf,
           W_conv, b_conv, bias):
    hi = hyperedge_index.astype(jnp.int32)
    row = hi[0]
    col = hi[1]

    W_big = jnp.kron(jnp.eye(D, dtype=W_conv.dtype), W_conv)
    b_big = jnp.tile(b_conv, D)
    Wp = jnp.pad(W_sheaf[:F], ((0, 0), (0, L - D)))
    bp = jnp.pad(b_sheaf, (0, L - D))
    Wg = jnp.pad(W_sheaf[F:], ((0, 0), (0, L - D)))
    bg = jnp.zeros((L,), jnp.float32).at[D].set(1.0)
    bias128 = jnp.tile(bias, D).reshape(1, F)

    xl, parr, garr = _prep(x, W_lin, b_lin, W_big, b_big, Wp, bp, Wg, bg)

    qc_parts, deg_parts = _sc_countsum(row, col, garr)
    qmean = _mid1(qc_parts)
    alpha = _sc_alpha(row, col, parr, qmean)

    m_parts = _sc_pass(row, col, alpha, xl)
    m_fin = _mid2(m_parts, qmean)
    y_parts = _sc_pass(col, row, alpha, m_fin)
    y = _final(y_parts, deg_parts, bias128)
    return y[:N]
